# baseline (speedup 1.0000x reference)
import jax
import jax.numpy as jnp
from jax.experimental import pallas as pl
from jax.experimental.pallas import tpu as pltpu

RADIUS = 0.2
C_IN = 32
C_OUT = 32
HIDDEN = 64

A_BLK = 256
B_BLK = 128


def _g_matmul(f_ref, w_ref, out_ref):
    out_ref[...] = jnp.dot(
        f_ref[...], w_ref[...],
        preferred_element_type=jnp.float32,
        precision=jax.lax.Precision.HIGHEST,
    )


def _conv_kernel(ga_ref, gbT_ref, w1rep_ref, g_ref, out_ref, hm_ref):
    bo = pl.program_id(2)
    ga = ga_ref[0]
    gbT = gbT_ref[0]
    w1rep = w1rep_ref[...]
    d0 = gbT[0:1, :] - ga[:, 0:1]
    d1 = gbT[1:2, :] - ga[:, 1:2]
    d2 = gbT[2:3, :] - ga[:, 2:3]
    n2 = d0 * d0 + d1 * d1 + d2 * d2
    m = (jnp.sqrt(n2) < RADIUS).astype(jnp.float32)
    gbrep = jnp.tile(gbT, (1, HIDDEN))
    bmat = jnp.concatenate([w1rep * gbrep, w1rep], axis=0)
    amat = jnp.concatenate([jnp.ones((A_BLK, 3), jnp.float32), -ga], axis=1)
    hm_ref[...] = jnp.dot(
        amat, bmat,
        preferred_element_type=jnp.float32,
        precision=jax.lax.Precision.HIGHEST,
    )
    for k in range(HIDDEN):
        sl = pl.ds(k * B_BLK, B_BLK)
        hm_ref[:, sl] = jnp.maximum(hm_ref[:, sl], 0.0) * m
    acc = jnp.dot(
        hm_ref[...], g_ref[0, 0],
        preferred_element_type=jnp.float32,
        precision=jax.lax.Precision.DEFAULT,
    )

    @pl.when(bo == 0)
    def _():
        out_ref[0] = acc

    @pl.when(bo != 0)
    def _():
        out_ref[0] = out_ref[0] + acc


def kernel(features, geometry, W1, W2):
    batch, n, _ = geometry.shape
    n_ao = n // A_BLK
    n_bo = n // B_BLK

    w2t = W2.reshape(HIDDEN, C_OUT, C_IN).transpose(2, 0, 1).reshape(C_IN, HIDDEN * C_OUT)
    g1 = pl.pallas_call(
        _g_matmul,
        out_shape=jax.ShapeDtypeStruct((batch * n, HIDDEN * C_OUT), jnp.float32),
    )(features.reshape(batch * n, C_IN), w2t)
    g = (
        g1.reshape(batch, n_bo, B_BLK, HIDDEN, C_OUT)
        .transpose(0, 1, 3, 2, 4)
        .reshape(batch, n_bo, HIDDEN * B_BLK, C_OUT)
    )

    gT = geometry.transpose(0, 2, 1)
    w1rep = jnp.repeat(W1.T.reshape(HIDDEN, 1, 3), B_BLK, axis=1).transpose(2, 0, 1).reshape(3, HIDDEN * B_BLK)

    out = pl.pallas_call(
        _conv_kernel,
        grid=(batch, n_ao, n_bo),
        in_specs=[
            pl.BlockSpec((1, A_BLK, 3), lambda z, ao, bo: (z, ao, 0)),
            pl.BlockSpec((1, 3, B_BLK), lambda z, ao, bo: (z, 0, bo)),
            pl.BlockSpec((3, HIDDEN * B_BLK), lambda z, ao, bo: (0, 0)),
            pl.BlockSpec((1, 1, HIDDEN * B_BLK, C_OUT), lambda z, ao, bo: (z, bo, 0, 0)),
        ],
        out_specs=pl.BlockSpec((1, A_BLK, C_OUT), lambda z, ao, bo: (z, ao, 0)),
        out_shape=jax.ShapeDtypeStruct((batch, n, C_OUT), jnp.float32),
        scratch_shapes=[pltpu.VMEM((A_BLK, HIDDEN * B_BLK), jnp.float32)],
    )(geometry, gT, w1rep, g)
    return out


# speedup vs baseline: 2.1182x; 2.1182x over previous
import jax
import jax.numpy as jnp
from jax.experimental import pallas as pl
from jax.experimental.pallas import tpu as pltpu

RADIUS = 0.2
C_IN = 32
C_OUT = 32
HIDDEN = 64

A_BLK = 512
B_BLK = 128


def _g_matmul(f_ref, w_ref, out_ref):
    out_ref[...] = jnp.dot(
        f_ref[...], w_ref[...],
        preferred_element_type=jnp.float32,
        precision=jax.lax.Precision.HIGHEST,
    ).astype(jnp.bfloat16)


def _conv_kernel(ga_ref, gbT_ref, w1_ref, w1T_ref, g_ref, out_ref, hm_ref):
    bo = pl.program_id(2)
    ga = ga_ref[0]
    gbT = gbT_ref[0]
    pa = jnp.dot(ga, w1_ref[...], preferred_element_type=jnp.float32,
                 precision=jax.lax.Precision.HIGHEST)
    pbT = jnp.dot(w1T_ref[...], gbT, preferred_element_type=jnp.float32,
                  precision=jax.lax.Precision.HIGHEST)
    d0 = gbT[0:1, :] - ga[:, 0:1]
    d1 = gbT[1:2, :] - ga[:, 1:2]
    d2 = gbT[2:3, :] - ga[:, 2:3]
    n2 = d0 * d0 + d1 * d1 + d2 * d2
    inmask = jnp.sqrt(n2) < RADIUS
    zero = jnp.zeros((), jnp.float32)
    for k in range(HIDDEN):
        hk = jnp.where(inmask, jnp.maximum(pbT[k:k + 1, :] - pa[:, k:k + 1], 0.0), zero)
        hm_ref[:, k * B_BLK:(k + 1) * B_BLK] = hk.astype(jnp.bfloat16)
    acc = jnp.dot(
        hm_ref[...], g_ref[0, 0],
        preferred_element_type=jnp.float32,
    )

    @pl.when(bo == 0)
    def _():
        out_ref[0] = acc

    @pl.when(bo != 0)
    def _():
        out_ref[0] = out_ref[0] + acc


def kernel(features, geometry, W1, W2):
    batch, n, _ = geometry.shape
    n_ao = n // A_BLK
    n_bo = n // B_BLK

    w2t = W2.reshape(HIDDEN, C_OUT, C_IN).transpose(2, 0, 1).reshape(C_IN, HIDDEN * C_OUT)
    g1 = pl.pallas_call(
        _g_matmul,
        out_shape=jax.ShapeDtypeStruct((batch * n, HIDDEN * C_OUT), jnp.bfloat16),
    )(features.reshape(batch * n, C_IN), w2t)
    g = (
        g1.reshape(batch, n_bo, B_BLK, HIDDEN, C_OUT)
        .transpose(0, 1, 3, 2, 4)
        .reshape(batch, n_bo, HIDDEN * B_BLK, C_OUT)
    )

    gT = geometry.transpose(0, 2, 1)
    w1T = W1.T

    out = pl.pallas_call(
        _conv_kernel,
        grid=(batch, n_ao, n_bo),
        in_specs=[
            pl.BlockSpec((1, A_BLK, 3), lambda z, ao, bo: (z, ao, 0)),
            pl.BlockSpec((1, 3, B_BLK), lambda z, ao, bo: (z, 0, bo)),
            pl.BlockSpec((3, HIDDEN), lambda z, ao, bo: (0, 0)),
            pl.BlockSpec((HIDDEN, 3), lambda z, ao, bo: (0, 0)),
            pl.BlockSpec((1, 1, HIDDEN * B_BLK, C_OUT), lambda z, ao, bo: (z, bo, 0, 0)),
        ],
        out_specs=pl.BlockSpec((1, A_BLK, C_OUT), lambda z, ao, bo: (z, ao, 0)),
        out_shape=jax.ShapeDtypeStruct((batch, n, C_OUT), jnp.float32),
        scratch_shapes=[pltpu.VMEM((A_BLK, HIDDEN * B_BLK), jnp.bfloat16)],
    )(geometry, gT, W1, w1T, g)
    return out


# speedup vs baseline: 3.2387x; 1.5290x over previous
import jax
import jax.numpy as jnp
from jax.experimental import pallas as pl
from jax.experimental.pallas import tpu as pltpu

RADIUS = 0.2
C_IN = 32
C_OUT = 32
HIDDEN = 64

B_BLK = 128


def _conv_kernel(ga_ref, gbT_ref, w1_ref, w1T_ref, fb_ref, w2t_ref,
                 out_ref, hm_ref, gs_ref):
    n = ga_ref.shape[1]
    bo = pl.program_id(1)
    ga = ga_ref[0]
    gbT = gbT_ref[0]
    gblk = jnp.dot(fb_ref[0], w2t_ref[...], preferred_element_type=jnp.float32)
    for k in range(HIDDEN):
        gs_ref[k * B_BLK:(k + 1) * B_BLK, :] = (
            gblk[:, k * C_OUT:(k + 1) * C_OUT].astype(jnp.bfloat16))
    pa = jnp.dot(ga, w1_ref[...], preferred_element_type=jnp.float32,
                 precision=jax.lax.Precision.HIGHEST)
    pbT = jnp.dot(w1T_ref[...], gbT, preferred_element_type=jnp.float32,
                  precision=jax.lax.Precision.HIGHEST)
    d0 = gbT[0:1, :] - ga[:, 0:1]
    d1 = gbT[1:2, :] - ga[:, 1:2]
    d2 = gbT[2:3, :] - ga[:, 2:3]
    n2 = d0 * d0 + d1 * d1 + d2 * d2
    inmask = jnp.sqrt(n2) < RADIUS
    zero = jnp.zeros((), jnp.float32)
    for k in range(HIDDEN):
        hk = jnp.where(inmask, jnp.maximum(pbT[k:k + 1, :] - pa[:, k:k + 1], 0.0), zero)
        hm_ref[:, k * B_BLK:(k + 1) * B_BLK] = hk.astype(jnp.bfloat16)
    acc = jnp.dot(
        hm_ref[...], gs_ref[...],
        preferred_element_type=jnp.float32,
    )

    @pl.when(bo == 0)
    def _():
        out_ref[0] = acc

    @pl.when(bo != 0)
    def _():
        out_ref[0] = out_ref[0] + acc


def kernel(features, geometry, W1, W2):
    batch, n, _ = geometry.shape
    n_bo = n // B_BLK

    w2t = (W2.reshape(HIDDEN, C_OUT, C_IN).transpose(2, 0, 1)
           .reshape(C_IN, HIDDEN * C_OUT).astype(jnp.bfloat16))
    fb = features.astype(jnp.bfloat16)
    gT = geometry.transpose(0, 2, 1)
    w1T = W1.T

    out = pl.pallas_call(
        _conv_kernel,
        grid=(batch, n_bo),
        in_specs=[
            pl.BlockSpec((1, n, 3), lambda z, bo: (z, 0, 0)),
            pl.BlockSpec((1, 3, B_BLK), lambda z, bo: (z, 0, bo)),
            pl.BlockSpec((3, HIDDEN), lambda z, bo: (0, 0)),
            pl.BlockSpec((HIDDEN, 3), lambda z, bo: (0, 0)),
            pl.BlockSpec((1, B_BLK, C_IN), lambda z, bo: (z, bo, 0)),
            pl.BlockSpec((C_IN, HIDDEN * C_OUT), lambda z, bo: (0, 0)),
        ],
        out_specs=pl.BlockSpec((1, n, C_OUT), lambda z, bo: (z, 0, 0)),
        out_shape=jax.ShapeDtypeStruct((batch, n, C_OUT), jnp.float32),
        scratch_shapes=[
            pltpu.VMEM((n, HIDDEN * B_BLK), jnp.bfloat16),
            pltpu.VMEM((HIDDEN * B_BLK, C_OUT), jnp.bfloat16),
        ],
    )(geometry, gT, W1, w1T, fb, w2t)
    return out


# speedup vs baseline: 3.6336x; 1.1219x over previous
import jax
import jax.numpy as jnp
from jax.experimental import pallas as pl
from jax.experimental.pallas import tpu as pltpu

RADIUS = 0.2
C_IN = 32
C_OUT = 32
HIDDEN = 64

B_BLK = 128


def _conv_kernel(gaT_ref, gb_ref, w1_ref, w1T_ref, fb_ref, w2t_ref,
                 out_ref, hm_ref, gs_ref):
    n = gaT_ref.shape[2]
    bo = pl.program_id(1)
    gaT = gaT_ref[0]
    gb = gb_ref[0]
    gblk = jnp.dot(fb_ref[0], w2t_ref[...], preferred_element_type=jnp.float32)
    for k in range(HIDDEN):
        gs_ref[k * B_BLK:(k + 1) * B_BLK, :] = (
            gblk[:, k * C_OUT:(k + 1) * C_OUT].astype(jnp.bfloat16))
    paT = jnp.dot(w1T_ref[...], gaT, preferred_element_type=jnp.float32,
                  precision=jax.lax.Precision.HIGHEST)
    pb = jnp.dot(gb, w1_ref[...], preferred_element_type=jnp.float32,
                 precision=jax.lax.Precision.HIGHEST)
    d0 = gaT[0:1, :] - gb[:, 0:1]
    d1 = gaT[1:2, :] - gb[:, 1:2]
    d2 = gaT[2:3, :] - gb[:, 2:3]
    n2 = d0 * d0 + d1 * d1 + d2 * d2
    inmask = jnp.sqrt(n2) < RADIUS
    zero = jnp.zeros((), jnp.float32)
    for k in range(HIDDEN):
        hk = jnp.where(inmask, jnp.maximum(pb[:, k:k + 1] - paT[k:k + 1, :], 0.0), zero)
        hm_ref[k * B_BLK:(k + 1) * B_BLK, :] = hk.astype(jnp.bfloat16)
    acc = jax.lax.dot_general(
        hm_ref[...], gs_ref[...],
        dimension_numbers=(((0,), (0,)), ((), ())),
        preferred_element_type=jnp.float32,
    )

    @pl.when(bo == 0)
    def _():
        out_ref[0] = acc

    @pl.when(bo != 0)
    def _():
        out_ref[0] = out_ref[0] + acc


def kernel(features, geometry, W1, W2):
    batch, n, _ = geometry.shape
    n_bo = n // B_BLK

    w2t = (W2.reshape(HIDDEN, C_OUT, C_IN).transpose(2, 0, 1)
           .reshape(C_IN, HIDDEN * C_OUT).astype(jnp.bfloat16))
    fb = features.astype(jnp.bfloat16)
    gT = geometry.transpose(0, 2, 1)
    w1T = W1.T

    out = pl.pallas_call(
        _conv_kernel,
        grid=(batch, n_bo),
        in_specs=[
            pl.BlockSpec((1, 3, n), lambda z, bo: (z, 0, 0)),
            pl.BlockSpec((1, B_BLK, 3), lambda z, bo: (z, bo, 0)),
            pl.BlockSpec((3, HIDDEN), lambda z, bo: (0, 0)),
            pl.BlockSpec((HIDDEN, 3), lambda z, bo: (0, 0)),
            pl.BlockSpec((1, B_BLK, C_IN), lambda z, bo: (z, bo, 0)),
            pl.BlockSpec((C_IN, HIDDEN * C_OUT), lambda z, bo: (0, 0)),
        ],
        out_specs=pl.BlockSpec((1, n, C_OUT), lambda z, bo: (z, 0, 0)),
        out_shape=jax.ShapeDtypeStruct((batch, n, C_OUT), jnp.float32),
        scratch_shapes=[
            pltpu.VMEM((HIDDEN * B_BLK, n), jnp.bfloat16),
            pltpu.VMEM((HIDDEN * B_BLK, C_OUT), jnp.bfloat16),
        ],
    )(gT, geometry, W1, w1T, fb, w2t)
    return out


# speedup vs baseline: 4.0754x; 1.1216x over previous
import jax
import jax.numpy as jnp
from jax.experimental import pallas as pl
from jax.experimental.pallas import tpu as pltpu

RADIUS = 0.2
C_IN = 32
C_OUT = 32
HIDDEN = 64


def _conv_kernel(gaT_ref, gb_ref, w1_ref, w1T_ref, fb_ref, w2t_ref,
                 out_ref, hm_ref, gs_ref):
    n = gaT_ref.shape[2]
    gaT = gaT_ref[0]
    gb = gb_ref[0]
    gblk = jnp.dot(fb_ref[0].astype(jnp.bfloat16), w2t_ref[...].astype(jnp.bfloat16),
                   preferred_element_type=jnp.float32)
    for k in range(HIDDEN):
        gs_ref[k * n:(k + 1) * n, :] = (
            gblk[:, k * C_OUT:(k + 1) * C_OUT].astype(jnp.bfloat16))
    paT = jnp.dot(w1T_ref[...], gaT, preferred_element_type=jnp.float32,
                  precision=jax.lax.Precision.HIGHEST)
    pb = jnp.dot(gb, w1_ref[...], preferred_element_type=jnp.float32,
                 precision=jax.lax.Precision.HIGHEST)
    d0 = gaT[0:1, :] - gb[:, 0:1]
    d1 = gaT[1:2, :] - gb[:, 1:2]
    d2 = gaT[2:3, :] - gb[:, 2:3]
    n2 = d0 * d0 + d1 * d1 + d2 * d2
    inmask = jnp.sqrt(n2) < RADIUS
    zero = jnp.zeros((), jnp.float32)
    for k in range(HIDDEN):
        hk = jnp.where(inmask, jnp.maximum(pb[:, k:k + 1] - paT[k:k + 1, :], 0.0), zero)
        hm_ref[k * n:(k + 1) * n, :] = hk.astype(jnp.bfloat16)
    out_ref[0] = jax.lax.dot_general(
        hm_ref[...], gs_ref[...],
        dimension_numbers=(((0,), (0,)), ((), ())),
        preferred_element_type=jnp.float32,
    )


def kernel(features, geometry, W1, W2):
    batch, n, _ = geometry.shape

    w2t = (W2.reshape(HIDDEN, C_OUT, C_IN).transpose(2, 0, 1)
           .reshape(C_IN, HIDDEN * C_OUT))
    gT = geometry.transpose(0, 2, 1)
    w1T = W1.T

    out = pl.pallas_call(
        _conv_kernel,
        grid=(batch,),
        in_specs=[
            pl.BlockSpec((1, 3, n), lambda z: (z, 0, 0)),
            pl.BlockSpec((1, n, 3), lambda z: (z, 0, 0)),
            pl.BlockSpec((3, HIDDEN), lambda z: (0, 0)),
            pl.BlockSpec((HIDDEN, 3), lambda z: (0, 0)),
            pl.BlockSpec((1, n, C_IN), lambda z: (z, 0, 0)),
            pl.BlockSpec((C_IN, HIDDEN * C_OUT), lambda z: (0, 0)),
        ],
        out_specs=pl.BlockSpec((1, n, C_OUT), lambda z: (z, 0, 0)),
        out_shape=jax.ShapeDtypeStruct((batch, n, C_OUT), jnp.float32),
        scratch_shapes=[
            pltpu.VMEM((HIDDEN * n, n), jnp.bfloat16),
            pltpu.VMEM((HIDDEN * n, C_OUT), jnp.bfloat16),
        ],
    )(gT, geometry, W1, w1T, features, w2t)
    return out
